# baseline (speedup 1.0000x reference)
import functools

import jax
import jax.numpy as jnp
from jax import lax
from jax.experimental import pallas as pl
from jax.experimental.pallas import tpu as pltpu
from jax.experimental.pallas import tpu_sc as plsc

D_MODEL = 128
SEQ_LEN = 200
BATCH = 1024
ROW = SEQ_LEN * D_MODEL

NC = 2
NS = 16
NW = NC * NS
LANES = 16

B_PER_W = BATCH // NW


TC_ROWS = 640
SC_ROWS = BATCH - TC_ROWS
SC_B_PER_W = SC_ROWS // NW

HALF = ROW // 2
NBUF = 8
LOOKAHEAD = 4
CHUNKS_PER_W = SC_B_PER_W * 2
RING_ITERS = CHUNKS_PER_W // NBUF


def _sc_add(x_flat, pe_flat):
    mesh = plsc.VectorSubcoreMesh(core_axis_name="c", subcore_axis_name="s")

    @functools.partial(
        pl.kernel,
        mesh=mesh,
        out_type=jax.ShapeDtypeStruct((SC_ROWS * ROW,), jnp.float32),
        scratch_types=[
            pltpu.VMEM((ROW,), jnp.float32),
            pltpu.VMEM((NBUF, HALF), jnp.float32),
            pltpu.SemaphoreType.DMA,
            pltpu.SemaphoreType.DMA,
        ],
    )
    def k(x_hbm, pe_hbm, out_hbm, pe_v, buf, sem_in, sem_out):
        wid = lax.axis_index("s") * NC + lax.axis_index("c")
        obase = wid * SC_B_PER_W * ROW
        ibase = TC_ROWS * ROW + obase
        pltpu.sync_copy(pe_hbm, pe_v)

        def in_copy(c, b):
            return pltpu.make_async_copy(
                x_hbm.at[pl.ds(ibase + c * HALF, HALF)], buf.at[b], sem_in)

        def out_copy(c, b):
            return pltpu.make_async_copy(
                buf.at[b], out_hbm.at[pl.ds(obase + c * HALF, HALF)], sem_out)

        for b in range(LOOKAHEAD):
            in_copy(b, b).start()

        def ring_step(g, _):
            for b in range(NBUF):
                c = g * NBUF + b
                pe_base = (b % 2) * HALF
                in_copy(c, b).wait()

                @plsc.parallel_loop(0, HALF, step=LANES, unroll=16)
                def _add(off):
                    plsc.addupdate(
                        buf.at[b, pl.ds(off, LANES)],
                        pe_v[pl.ds(pe_base + off, LANES)],
                    )

                out_copy(c, b).start()

                @pl.when(c + LOOKAHEAD < CHUNKS_PER_W)
                def _prefetch_next_in():
                    @pl.when(c >= NBUF - LOOKAHEAD)
                    def _drain_prev_out():
                        out_copy(c + LOOKAHEAD - NBUF,
                                 (b + LOOKAHEAD) % NBUF).wait()
                    in_copy(c + LOOKAHEAD, (b + LOOKAHEAD) % NBUF).start()
            return 0

        lax.fori_loop(0, RING_ITERS, ring_step, 0)
        for c in range(CHUNKS_PER_W - LOOKAHEAD, CHUNKS_PER_W):
            out_copy(c, c % NBUF).wait()

    return k(x_flat, pe_flat)


B_BLK = 16


def _tc_body(x_ref, pe_ref, o_ref):
    o_ref[...] = x_ref[...] + pe_ref[...]


def _tc_add(x, pe):
    seq_len, d = pe.shape[1], pe.shape[2]
    return pl.pallas_call(
        _tc_body,
        grid=(TC_ROWS // B_BLK,),
        in_specs=[
            pl.BlockSpec((B_BLK, seq_len, d), lambda i: (i, 0, 0)),
            pl.BlockSpec((1, seq_len, d), lambda i: (0, 0, 0)),
        ],
        out_specs=pl.BlockSpec((B_BLK, seq_len, d), lambda i: (i, 0, 0)),
        out_shape=jax.ShapeDtypeStruct((TC_ROWS, seq_len, d), x.dtype),
        compiler_params=pltpu.CompilerParams(
            dimension_semantics=("arbitrary",),
        ),
    )(x, pe)


def kernel(x, seq_lens, pos_embedding):
    del seq_lens
    batch, seq_len, d = x.shape
    x_flat = x.reshape(batch * seq_len * d)
    pe = pos_embedding[:seq_len][None, :, :]
    pe_flat = pe.reshape(seq_len * d)
    out_sc = _sc_add(x_flat, pe_flat)
    out_tc = _tc_add(x, pe)
    return jnp.concatenate(
        [out_tc, out_sc.reshape(SC_ROWS, seq_len, d)], axis=0)


# speedup vs baseline: 1.0619x; 1.0619x over previous
import functools

import jax
import jax.numpy as jnp
from jax import lax
from jax.experimental import pallas as pl
from jax.experimental.pallas import tpu as pltpu
from jax.experimental.pallas import tpu_sc as plsc

D_MODEL = 128
SEQ_LEN = 200
BATCH = 1024
ROW = SEQ_LEN * D_MODEL

NC = 2
NS = 16
NW = NC * NS
LANES = 16

B_PER_W = BATCH // NW

NBUF = 4
LOOKAHEAD = 2
CHUNKS_PER_W = B_PER_W
RING_ITERS = CHUNKS_PER_W // NBUF


def _sc_add(x_flat, pe_flat):
    mesh = plsc.VectorSubcoreMesh(core_axis_name="c", subcore_axis_name="s")

    @functools.partial(
        pl.kernel,
        mesh=mesh,
        out_type=jax.ShapeDtypeStruct((BATCH * ROW,), jnp.float32),
        scratch_types=[
            pltpu.VMEM((ROW,), jnp.float32),
            pltpu.VMEM((NBUF, ROW), jnp.float32),
            pltpu.SemaphoreType.DMA,
            pltpu.SemaphoreType.DMA,
        ],
    )
    def k(x_hbm, pe_hbm, out_hbm, pe_v, buf, sem_in, sem_out):
        wid = lax.axis_index("s") * NC + lax.axis_index("c")
        base = wid * B_PER_W * ROW
        pltpu.sync_copy(pe_hbm, pe_v)

        def in_copy(c, b):
            return pltpu.make_async_copy(
                x_hbm.at[pl.ds(base + c * ROW, ROW)], buf.at[b], sem_in)

        def out_copy(c, b):
            return pltpu.make_async_copy(
                buf.at[b], out_hbm.at[pl.ds(base + c * ROW, ROW)], sem_out)

        for b in range(LOOKAHEAD):
            in_copy(b, b).start()

        def ring_step(g, _):
            for b in range(NBUF):
                c = g * NBUF + b
                in_copy(c, b).wait()

                @plsc.parallel_loop(0, ROW, step=LANES, unroll=16)
                def _add(off):
                    plsc.addupdate(
                        buf.at[b, pl.ds(off, LANES)],
                        pe_v[pl.ds(off, LANES)],
                    )

                out_copy(c, b).start()

                @pl.when(c + LOOKAHEAD < CHUNKS_PER_W)
                def _prefetch_next_in():
                    @pl.when(c >= NBUF - LOOKAHEAD)
                    def _drain_prev_out():
                        out_copy(c + LOOKAHEAD - NBUF,
                                 (b + LOOKAHEAD) % NBUF).wait()
                    in_copy(c + LOOKAHEAD, (b + LOOKAHEAD) % NBUF).start()
            return 0

        lax.fori_loop(0, RING_ITERS, ring_step, 0)
        for c in range(CHUNKS_PER_W - LOOKAHEAD, CHUNKS_PER_W):
            out_copy(c, c % NBUF).wait()

    return k(x_flat, pe_flat)


def kernel(x, seq_lens, pos_embedding):
    del seq_lens
    batch, seq_len, d = x.shape
    x_flat = x.reshape(batch * seq_len * d)
    pe_flat = pos_embedding[:seq_len].reshape(seq_len * d)
    out = _sc_add(x_flat, pe_flat)
    return out.reshape(batch, seq_len, d)


# speedup vs baseline: 1.3339x; 1.2561x over previous
import functools

import jax
import jax.numpy as jnp
from jax import lax
from jax.experimental import pallas as pl
from jax.experimental.pallas import tpu as pltpu
from jax.experimental.pallas import tpu_sc as plsc

D_MODEL = 128
SEQ_LEN = 200
BATCH = 1024
ROW = SEQ_LEN * D_MODEL

NC = 2
NS = 16
NW = NC * NS
LANES = 16

TC_ROWS = 640
SC_ROWS = BATCH - TC_ROWS
SC_B_PER_W = SC_ROWS // NW

NBUF = 4
LOOKAHEAD = 2
CHUNKS_PER_W = SC_B_PER_W
RING_ITERS = CHUNKS_PER_W // NBUF


def _sc_add(x_flat, pe_flat):
    mesh = plsc.VectorSubcoreMesh(core_axis_name="c", subcore_axis_name="s")

    @functools.partial(
        pl.kernel,
        mesh=mesh,
        out_type=jax.ShapeDtypeStruct((SC_ROWS * ROW,), jnp.float32),
        scratch_types=[
            pltpu.VMEM((ROW,), jnp.float32),
            pltpu.VMEM((NBUF, ROW), jnp.float32),
            pltpu.SemaphoreType.DMA,
            pltpu.SemaphoreType.DMA,
        ],
    )
    def k(x_hbm, pe_hbm, out_hbm, pe_v, buf, sem_in, sem_out):
        wid = lax.axis_index("s") * NC + lax.axis_index("c")
        obase = wid * SC_B_PER_W * ROW
        ibase = TC_ROWS * ROW + obase
        pltpu.sync_copy(pe_hbm, pe_v)

        def in_copy(c, b):
            return pltpu.make_async_copy(
                x_hbm.at[pl.ds(ibase + c * ROW, ROW)], buf.at[b], sem_in)

        def out_copy(c, b):
            return pltpu.make_async_copy(
                buf.at[b], out_hbm.at[pl.ds(obase + c * ROW, ROW)], sem_out)

        for b in range(LOOKAHEAD):
            in_copy(b, b).start()

        def ring_step(g, _):
            for b in range(NBUF):
                c = g * NBUF + b
                in_copy(c, b).wait()

                @plsc.parallel_loop(0, ROW, step=LANES, unroll=16)
                def _add(off):
                    plsc.addupdate(
                        buf.at[b, pl.ds(off, LANES)],
                        pe_v[pl.ds(off, LANES)],
                    )

                out_copy(c, b).start()

                @pl.when(c + LOOKAHEAD < CHUNKS_PER_W)
                def _prefetch_next_in():
                    @pl.when(c >= NBUF - LOOKAHEAD)
                    def _drain_prev_out():
                        out_copy(c + LOOKAHEAD - NBUF,
                                 (b + LOOKAHEAD) % NBUF).wait()
                    in_copy(c + LOOKAHEAD, (b + LOOKAHEAD) % NBUF).start()
            return 0

        lax.fori_loop(0, RING_ITERS, ring_step, 0)
        for c in range(CHUNKS_PER_W - LOOKAHEAD, CHUNKS_PER_W):
            out_copy(c, c % NBUF).wait()

    return k(x_flat, pe_flat)


B_BLK = 16


def _tc_body(x_ref, pe_ref, o_ref):
    o_ref[...] = x_ref[...] + pe_ref[...]


def _tc_add(x, pe):
    batch, seq_len, d = x.shape
    return pl.pallas_call(
        _tc_body,
        grid=(TC_ROWS // B_BLK,),
        in_specs=[
            pl.BlockSpec((B_BLK, seq_len, d), lambda i: (i, 0, 0)),
            pl.BlockSpec((1, seq_len, d), lambda i: (0, 0, 0)),
        ],
        out_specs=pl.BlockSpec((B_BLK, seq_len, d), lambda i: (i, 0, 0)),
        out_shape=jax.ShapeDtypeStruct((batch, seq_len, d), x.dtype),
        compiler_params=pltpu.CompilerParams(
            dimension_semantics=("arbitrary",),
        ),
    )(x, pe)


def kernel(x, seq_lens, pos_embedding):
    del seq_lens
    batch, seq_len, d = x.shape
    x_flat = x.reshape(batch * seq_len * d)
    pe = pos_embedding[:seq_len][None, :, :]
    pe_flat = pe.reshape(seq_len * d)
    out_sc = _sc_add(x_flat, pe_flat)
    out_full = _tc_add(x, pe)
    return lax.dynamic_update_slice(
        out_full, out_sc.reshape(SC_ROWS, seq_len, d), (TC_ROWS, 0, 0))


# speedup vs baseline: 1.4028x; 1.0516x over previous
import functools

import jax
import jax.numpy as jnp
from jax import lax
from jax.experimental import pallas as pl
from jax.experimental.pallas import tpu as pltpu
from jax.experimental.pallas import tpu_sc as plsc

D_MODEL = 128
SEQ_LEN = 200
BATCH = 1024
ROW = SEQ_LEN * D_MODEL

NC = 2
NS = 16
NW = NC * NS
LANES = 16

TC_ROWS = 768
SC_ROWS = BATCH - TC_ROWS
SC_B_PER_W = SC_ROWS // NW

NBUF = 4
LOOKAHEAD = 2
CHUNKS_PER_W = SC_B_PER_W
RING_ITERS = CHUNKS_PER_W // NBUF


def _sc_add(x_flat, pe_flat):
    mesh = plsc.VectorSubcoreMesh(core_axis_name="c", subcore_axis_name="s")

    @functools.partial(
        pl.kernel,
        mesh=mesh,
        out_type=jax.ShapeDtypeStruct((SC_ROWS * ROW,), jnp.float32),
        scratch_types=[
            pltpu.VMEM((ROW,), jnp.float32),
            pltpu.VMEM((NBUF, ROW), jnp.float32),
            pltpu.SemaphoreType.DMA,
            pltpu.SemaphoreType.DMA,
        ],
    )
    def k(x_hbm, pe_hbm, out_hbm, pe_v, buf, sem_in, sem_out):
        wid = lax.axis_index("s") * NC + lax.axis_index("c")
        obase = wid * SC_B_PER_W * ROW
        ibase = TC_ROWS * ROW + obase
        pltpu.sync_copy(pe_hbm, pe_v)

        def in_copy(c, b):
            return pltpu.make_async_copy(
                x_hbm.at[pl.ds(ibase + c * ROW, ROW)], buf.at[b], sem_in)

        def out_copy(c, b):
            return pltpu.make_async_copy(
                buf.at[b], out_hbm.at[pl.ds(obase + c * ROW, ROW)], sem_out)

        for b in range(LOOKAHEAD):
            in_copy(b, b).start()

        def ring_step(g, _):
            for b in range(NBUF):
                c = g * NBUF + b
                in_copy(c, b).wait()

                @plsc.parallel_loop(0, ROW, step=LANES, unroll=16)
                def _add(off):
                    plsc.addupdate(
                        buf.at[b, pl.ds(off, LANES)],
                        pe_v[pl.ds(off, LANES)],
                    )

                out_copy(c, b).start()

                @pl.when(c + LOOKAHEAD < CHUNKS_PER_W)
                def _prefetch_next_in():
                    @pl.when(c >= NBUF - LOOKAHEAD)
                    def _drain_prev_out():
                        out_copy(c + LOOKAHEAD - NBUF,
                                 (b + LOOKAHEAD) % NBUF).wait()
                    in_copy(c + LOOKAHEAD, (b + LOOKAHEAD) % NBUF).start()
            return 0

        lax.fori_loop(0, RING_ITERS, ring_step, 0)
        for c in range(CHUNKS_PER_W - LOOKAHEAD, CHUNKS_PER_W):
            out_copy(c, c % NBUF).wait()

    return k(x_flat, pe_flat)


B_BLK = 16


def _tc_body(x_ref, pe_ref, o_ref):
    o_ref[...] = x_ref[...] + pe_ref[...]


def _tc_add(x, pe):
    batch, seq_len, d = x.shape
    return pl.pallas_call(
        _tc_body,
        grid=(TC_ROWS // B_BLK,),
        in_specs=[
            pl.BlockSpec((B_BLK, seq_len, d), lambda i: (i, 0, 0)),
            pl.BlockSpec((1, seq_len, d), lambda i: (0, 0, 0)),
        ],
        out_specs=pl.BlockSpec((B_BLK, seq_len, d), lambda i: (i, 0, 0)),
        out_shape=jax.ShapeDtypeStruct((batch, seq_len, d), x.dtype),
        compiler_params=pltpu.CompilerParams(
            dimension_semantics=("arbitrary",),
        ),
    )(x, pe)


def kernel(x, seq_lens, pos_embedding):
    del seq_lens
    batch, seq_len, d = x.shape
    x_flat = x.reshape(batch * seq_len * d)
    pe = pos_embedding[:seq_len][None, :, :]
    pe_flat = pe.reshape(seq_len * d)
    out_sc = _sc_add(x_flat, pe_flat)
    out_full = _tc_add(x, pe)
    return lax.dynamic_update_slice(
        out_full, out_sc.reshape(SC_ROWS, seq_len, d), (TC_ROWS, 0, 0))


# speedup vs baseline: 1.4749x; 1.0515x over previous
import functools

import jax
import jax.numpy as jnp
from jax import lax
from jax.experimental import pallas as pl
from jax.experimental.pallas import tpu as pltpu
from jax.experimental.pallas import tpu_sc as plsc

D_MODEL = 128
SEQ_LEN = 200
BATCH = 1024
ROW = SEQ_LEN * D_MODEL

NC = 2
NS = 16
NW = NC * NS
LANES = 16

TC_ROWS = 896
SC_ROWS = BATCH - TC_ROWS
SC_B_PER_W = SC_ROWS // NW

NBUF = 4
LOOKAHEAD = 2
CHUNKS_PER_W = SC_B_PER_W
RING_ITERS = CHUNKS_PER_W // NBUF


def _sc_add(x_flat, pe_flat):
    mesh = plsc.VectorSubcoreMesh(core_axis_name="c", subcore_axis_name="s")

    @functools.partial(
        pl.kernel,
        mesh=mesh,
        out_type=jax.ShapeDtypeStruct((SC_ROWS * ROW,), jnp.float32),
        scratch_types=[
            pltpu.VMEM((ROW,), jnp.float32),
            pltpu.VMEM((NBUF, ROW), jnp.float32),
            pltpu.SemaphoreType.DMA,
            pltpu.SemaphoreType.DMA,
        ],
    )
    def k(x_hbm, pe_hbm, out_hbm, pe_v, buf, sem_in, sem_out):
        wid = lax.axis_index("s") * NC + lax.axis_index("c")
        obase = wid * SC_B_PER_W * ROW
        ibase = TC_ROWS * ROW + obase
        pltpu.sync_copy(pe_hbm, pe_v)

        def in_copy(c, b):
            return pltpu.make_async_copy(
                x_hbm.at[pl.ds(ibase + c * ROW, ROW)], buf.at[b], sem_in)

        def out_copy(c, b):
            return pltpu.make_async_copy(
                buf.at[b], out_hbm.at[pl.ds(obase + c * ROW, ROW)], sem_out)

        for b in range(LOOKAHEAD):
            in_copy(b, b).start()

        def ring_step(g, _):
            for b in range(NBUF):
                c = g * NBUF + b
                in_copy(c, b).wait()

                @plsc.parallel_loop(0, ROW, step=LANES, unroll=16)
                def _add(off):
                    plsc.addupdate(
                        buf.at[b, pl.ds(off, LANES)],
                        pe_v[pl.ds(off, LANES)],
                    )

                out_copy(c, b).start()

                @pl.when(c + LOOKAHEAD < CHUNKS_PER_W)
                def _prefetch_next_in():
                    @pl.when(c >= NBUF - LOOKAHEAD)
                    def _drain_prev_out():
                        out_copy(c + LOOKAHEAD - NBUF,
                                 (b + LOOKAHEAD) % NBUF).wait()
                    in_copy(c + LOOKAHEAD, (b + LOOKAHEAD) % NBUF).start()
            return 0

        lax.fori_loop(0, RING_ITERS, ring_step, 0)
        for c in range(CHUNKS_PER_W - LOOKAHEAD, CHUNKS_PER_W):
            out_copy(c, c % NBUF).wait()

    return k(x_flat, pe_flat)


B_BLK = 16


def _tc_body(x_ref, pe_ref, o_ref):
    o_ref[...] = x_ref[...] + pe_ref[...]


def _tc_add(x, pe):
    batch, seq_len, d = x.shape
    return pl.pallas_call(
        _tc_body,
        grid=(TC_ROWS // B_BLK,),
        in_specs=[
            pl.BlockSpec((B_BLK, seq_len, d), lambda i: (i, 0, 0)),
            pl.BlockSpec((1, seq_len, d), lambda i: (0, 0, 0)),
        ],
        out_specs=pl.BlockSpec((B_BLK, seq_len, d), lambda i: (i, 0, 0)),
        out_shape=jax.ShapeDtypeStruct((batch, seq_len, d), x.dtype),
        compiler_params=pltpu.CompilerParams(
            dimension_semantics=("arbitrary",),
        ),
    )(x, pe)


def kernel(x, seq_lens, pos_embedding):
    del seq_lens
    batch, seq_len, d = x.shape
    x_flat = x.reshape(batch * seq_len * d)
    pe = pos_embedding[:seq_len][None, :, :]
    pe_flat = pe.reshape(seq_len * d)
    out_sc = _sc_add(x_flat, pe_flat)
    out_full = _tc_add(x, pe)
    return lax.dynamic_update_slice(
        out_full, out_sc.reshape(SC_ROWS, seq_len, d), (TC_ROWS, 0, 0))


# speedup vs baseline: 1.4829x; 1.0054x over previous
import functools

import jax
import jax.numpy as jnp
from jax import lax
from jax.experimental import pallas as pl
from jax.experimental.pallas import tpu as pltpu
from jax.experimental.pallas import tpu_sc as plsc

D_MODEL = 128
SEQ_LEN = 200
BATCH = 1024
ROW = SEQ_LEN * D_MODEL

NC = 2
NS = 16
NW = NC * NS
LANES = 16

TC_ROWS = 896
SC_ROWS = BATCH - TC_ROWS
SC_B_PER_W = SC_ROWS // NW

NBUF = 4
LOOKAHEAD = 2
CHUNKS_PER_W = SC_B_PER_W
RING_ITERS = CHUNKS_PER_W // NBUF


def _sc_add(x_flat, pe_flat):
    mesh = plsc.VectorSubcoreMesh(core_axis_name="c", subcore_axis_name="s")

    @functools.partial(
        pl.kernel,
        mesh=mesh,
        out_type=jax.ShapeDtypeStruct((SC_ROWS * ROW,), jnp.float32),
        scratch_types=[
            pltpu.VMEM((ROW,), jnp.float32),
            pltpu.VMEM((NBUF, ROW), jnp.float32),
            pltpu.SemaphoreType.DMA,
            pltpu.SemaphoreType.DMA,
        ],
    )
    def k(x_hbm, pe_hbm, out_hbm, pe_v, buf, sem_in, sem_out):
        wid = lax.axis_index("s") * NC + lax.axis_index("c")
        obase = wid * SC_B_PER_W * ROW
        ibase = TC_ROWS * ROW + obase
        pltpu.sync_copy(pe_hbm, pe_v)

        def in_copy(c, b):
            return pltpu.make_async_copy(
                x_hbm.at[pl.ds(ibase + c * ROW, ROW)], buf.at[b], sem_in)

        def out_copy(c, b):
            return pltpu.make_async_copy(
                buf.at[b], out_hbm.at[pl.ds(obase + c * ROW, ROW)], sem_out)

        for b in range(LOOKAHEAD):
            in_copy(b, b).start()

        def ring_step(g, _):
            for b in range(NBUF):
                c = g * NBUF + b
                in_copy(c, b).wait()

                @plsc.parallel_loop(0, ROW, step=LANES, unroll=16)
                def _add(off):
                    plsc.addupdate(
                        buf.at[b, pl.ds(off, LANES)],
                        pe_v[pl.ds(off, LANES)],
                    )

                out_copy(c, b).start()

                @pl.when(c + LOOKAHEAD < CHUNKS_PER_W)
                def _prefetch_next_in():
                    @pl.when(c >= NBUF - LOOKAHEAD)
                    def _drain_prev_out():
                        out_copy(c + LOOKAHEAD - NBUF,
                                 (b + LOOKAHEAD) % NBUF).wait()
                    in_copy(c + LOOKAHEAD, (b + LOOKAHEAD) % NBUF).start()
            return 0

        lax.fori_loop(0, RING_ITERS, ring_step, 0)
        for c in range(CHUNKS_PER_W - NBUF, CHUNKS_PER_W):
            out_copy(c, c % NBUF).wait()

    return k(x_flat, pe_flat)


B_BLK = 16


def _tc_body(x_ref, pe_ref, o_ref):
    o_ref[...] = x_ref[...] + pe_ref[...]


def _tc_add(x, pe):
    batch, seq_len, d = x.shape
    return pl.pallas_call(
        _tc_body,
        grid=(TC_ROWS // B_BLK,),
        in_specs=[
            pl.BlockSpec((B_BLK, seq_len, d), lambda i: (i, 0, 0)),
            pl.BlockSpec((1, seq_len, d), lambda i: (0, 0, 0)),
        ],
        out_specs=pl.BlockSpec((B_BLK, seq_len, d), lambda i: (i, 0, 0)),
        out_shape=jax.ShapeDtypeStruct((batch, seq_len, d), x.dtype),
        compiler_params=pltpu.CompilerParams(
            dimension_semantics=("arbitrary",),
        ),
    )(x, pe)


def kernel(x, seq_lens, pos_embedding):
    del seq_lens
    batch, seq_len, d = x.shape
    x_flat = x.reshape(batch * seq_len * d)
    pe = pos_embedding[:seq_len][None, :, :]
    pe_flat = pe.reshape(seq_len * d)
    out_sc = _sc_add(x_flat, pe_flat)
    out_full = _tc_add(x, pe)
    return lax.dynamic_update_slice(
        out_full, out_sc.reshape(SC_ROWS, seq_len, d), (TC_ROWS, 0, 0))
